# baseline (speedup 1.0000x reference)
import functools

import jax
import jax.numpy as jnp
from jax import lax
from jax.experimental import pallas as pl
from jax.experimental.pallas import tpu as pltpu
from jax.experimental.pallas import tpu_sc as plsc

B = 4096
H = 256
TAGS = 19
N_BAGS = 3 * B
TOTAL = 73728
CH = 48
NW = 32
BAGS_PER_W = N_BAGS // NW
TRASH = BAGS_PER_W
NOFF = BAGS_PER_W // 16


def _sc_embed(cid_hbm, off_hbm, wf_hbm, tag_ids_hbm,
              ctab_hbm, ttab_hbm, wtab_hbm, x_hbm,
              idsbuf, idsbuf2, tbuf, rows, rowsb, offbuf, mbuf, acc, gsem, gsemb):
    c = lax.axis_index("c")
    s = lax.axis_index("s")
    w = c * 16 + s
    bag0 = w * BAGS_PER_W
    lanes = lax.iota(jnp.int32, 16)

    for k in range(BAGS_PER_W // CH):
        pltpu.sync_copy(wf_hbm.at[pl.ds(bag0 + k * CH, CH)], idsbuf)
        pltpu.async_copy(wtab_hbm.at[idsbuf],
                         acc.at[pl.ds(k * CH, CH)], gsem).wait()

    for k in range(BAGS_PER_W // 3 // 32):
        pltpu.sync_copy(tag_ids_hbm.at[pl.ds(w * (BAGS_PER_W // 3) + k * 32, 32)],
                        tbuf)
        pltpu.async_copy(ttab_hbm.at[tbuf], rows.at[pl.ds(0, 32)], gsem).wait()

        def tag_add(t, carry):
            r = 3 * (k * 32 + t)
            for q in range(H // 16):
                sl = pl.ds(q * 16, 16)
                acc[r, sl] = acc[r, sl] + rows[t, sl]
            return carry

        lax.fori_loop(0, 32, tag_add, 0)

    pltpu.sync_copy(off_hbm.at[pl.ds(bag0, 400)], offbuf)
    p_start = offbuf[pl.ds(0, 16)][0]
    p_end = offbuf[pl.ds(BAGS_PER_W, 16)][0]
    p0 = (p_start // 8) * 8
    nch = (p_end - p0 + CH - 1) // CH

    ids2 = [idsbuf, idsbuf2]
    rows2 = [rows, rowsb]
    sem2 = [gsem, gsemb]

    def issue(kk, par):
        pltpu.sync_copy(cid_hbm.at[pl.ds(p0 + kk * CH, CH)], ids2[par])
        pltpu.async_copy(ctab_hbm.at[ids2[par]], rows2[par], sem2[par])

    @pl.when(nch > 0)
    def _():
        issue(0, 0)

    NQ = H // 16
    mbuf[pl.ds(CH, 16)] = jnp.full((16,), -1, jnp.int32)

    def chunk(kk, carry):
        base = p0 + kk * CH

        for j in range(CH // 16):
            mbuf[pl.ds(j * 16, 16)] = jnp.zeros((16,), jnp.int32)
        for j in range(NOFF):
            offv = offbuf[pl.ds(j * 16, 16)]
            offn = offbuf[pl.ds(j * 16 + 1, 16)]
            m = (offv >= base) & (offv < base + CH) & (offv < offn)
            plsc.store_scatter(mbuf, [offv - base], 16 * j + lanes, mask=m)

        def segfin(j, cr):
            pos = base + j * 16 + lanes
            mv = jnp.maximum(plsc.cummax(mbuf[pl.ds(j * 16, 16)]), cr)
            newcr = mv[15]
            valid = (pos >= p_start) & (pos < p_end)
            mbuf[pl.ds(j * 16, 16)] = jnp.where(valid, mv, TRASH)
            return newcr

        cr = lax.fori_loop(0, CH // 16, segfin, carry)

        par = lax.rem(kk, 2)

        @pl.when(kk + 1 < nch)
        def _():
            @pl.when(par == 0)
            def _():
                issue(kk + 1, 1)

            @pl.when(par == 1)
            def _():
                issue(kk + 1, 0)

        def rmw(p):
            pltpu.make_async_copy(
                ctab_hbm.at[pl.ds(0, CH)], rows2[p], sem2[p]).wait()

            zeros = tuple(jnp.zeros((16,), jnp.float32) for _ in range(NQ))

            def inner(j, accs):
                sv = mbuf[pl.ds(j * 16, 16)]
                nxt = mbuf[pl.ds(j * 16 + 1, 16)]
                lastv = (sv != nxt).astype(jnp.int32)
                for l in range(16):
                    t = j * 16 + l
                    accs = tuple(a + rows2[p][t, pl.ds(q * 16, 16)]
                                 for q, a in enumerate(accs))
                    last = lastv[l] != 0
                    sg = sv[l]

                    @pl.when(last)
                    def _(accs=accs, sg=sg):
                        for q in range(NQ):
                            sl = pl.ds(q * 16, 16)
                            acc[sg, sl] = acc[sg, sl] + accs[q]

                    accs = tuple(jnp.where(last, z, a)
                                 for z, a in zip(zeros, accs))
                return accs

            lax.fori_loop(0, CH // 16, inner, zeros)

        @pl.when(par == 0)
        def _():
            rmw(0)

        @pl.when(par == 1)
        def _():
            rmw(1)

        return cr

    lax.fori_loop(0, nch, chunk, 0)

    pltpu.sync_copy(acc.at[pl.ds(0, BAGS_PER_W)],
                    x_hbm.at[pl.ds(bag0, BAGS_PER_W)])


_sc_embed_call = functools.partial(
    pl.kernel,
    out_type=jax.ShapeDtypeStruct((N_BAGS, H), jnp.float32),
    mesh=plsc.VectorSubcoreMesh(core_axis_name="c", subcore_axis_name="s"),
    compiler_params=pltpu.CompilerParams(needs_layout_passes=False),
    scratch_types=[
        pltpu.VMEM((CH,), jnp.int32),
        pltpu.VMEM((CH,), jnp.int32),
        pltpu.VMEM((32,), jnp.int32),
        pltpu.VMEM((CH, H), jnp.float32),
        pltpu.VMEM((CH, H), jnp.float32),
        pltpu.VMEM((400,), jnp.int32),
        pltpu.VMEM((CH + 16,), jnp.int32),
        pltpu.VMEM((BAGS_PER_W + 1, H), jnp.float32),
        pltpu.SemaphoreType.DMA,
        pltpu.SemaphoreType.DMA,
    ],
)(_sc_embed)


def _tc_classifier(x_ref, w_ref, b_ref, o_ref):
    y = jnp.dot(x_ref[...], w_ref[...], preferred_element_type=jnp.float32)
    y = jnp.maximum(y + b_ref[...], 0.0)
    y = y - jnp.max(y, axis=1, keepdims=True)
    o_ref[...] = jnp.exp(y)


def kernel(char_ids, offsets, prev_tag_ids, word_ids,
           char_table, tag_table, word_table, W_w, W_b):
    cid_pad = jnp.concatenate(
        [char_ids.astype(jnp.int32), jnp.zeros((CH,), jnp.int32)])
    off_pad = jnp.concatenate(
        [offsets.astype(jnp.int32), jnp.full((16,), TOTAL, jnp.int32)])
    wf = word_ids.reshape(-1).astype(jnp.int32)

    x = _sc_embed_call(cid_pad, off_pad, wf,
                       prev_tag_ids.astype(jnp.int32),
                       char_table, tag_table, word_table)
    x = x.reshape(B, 3 * H)

    blk = 512
    out = pl.pallas_call(
        _tc_classifier,
        grid=(B // blk,),
        in_specs=[
            pl.BlockSpec((blk, 3 * H), lambda i: (i, 0)),
            pl.BlockSpec((3 * H, TAGS), lambda i: (0, 0)),
            pl.BlockSpec((1, TAGS), lambda i: (0, 0)),
        ],
        out_specs=pl.BlockSpec((blk, TAGS), lambda i: (i, 0)),
        out_shape=jax.ShapeDtypeStruct((B, TAGS), jnp.float32),
    )(x, W_w.T, W_b.reshape(1, TAGS))
    return out


# speedup vs baseline: 1.1662x; 1.1662x over previous
import functools

import jax
import jax.numpy as jnp
from jax import lax
from jax.experimental import pallas as pl
from jax.experimental.pallas import tpu as pltpu
from jax.experimental.pallas import tpu_sc as plsc

B = 4096
H = 256
TAGS = 19
N_BAGS = 3 * B
TOTAL = 73728
CH = 48
NW = 32
BAGS_PER_W = N_BAGS // NW
TRASH = BAGS_PER_W
NOFF = BAGS_PER_W // 16


def _sc_embed(cid_hbm, off_hbm, wf_hbm, tag_ids_hbm,
              ctab_hbm, ttab_hbm, wtab_hbm, x_hbm,
              idsbuf, idsbuf2, tbuf, rows, rowsb, offbuf, mbuf, acc, gsem, gsemb):
    c = lax.axis_index("c")
    s = lax.axis_index("s")
    w = c * 16 + s
    bag0 = w * BAGS_PER_W
    lanes = lax.iota(jnp.int32, 16)

    for k in range(BAGS_PER_W // CH):
        pltpu.sync_copy(wf_hbm.at[pl.ds(bag0 + k * CH, CH)], idsbuf)
        pltpu.async_copy(wtab_hbm.at[idsbuf],
                         acc.at[pl.ds(k * CH, CH)], gsem).wait()

    for k in range(BAGS_PER_W // 3 // 32):
        pltpu.sync_copy(tag_ids_hbm.at[pl.ds(w * (BAGS_PER_W // 3) + k * 32, 32)],
                        tbuf)
        pltpu.async_copy(ttab_hbm.at[tbuf], rows.at[pl.ds(0, 32)], gsem).wait()

        def tag_add(t, carry):
            r = 3 * (k * 32 + t)
            for hh in range(H // 32):
                vi = rows[t, pl.ds(hh * 16, 16)]
                lo = plsc.bitcast(vi << 16, jnp.float32)
                hi = plsc.bitcast(vi & jnp.int32(-65536), jnp.float32)
                sl0 = pl.ds(hh * 32, 16)
                sl1 = pl.ds(hh * 32 + 16, 16)
                acc[r, sl0] = acc[r, sl0] + lo
                acc[r, sl1] = acc[r, sl1] + hi
            return carry

        lax.fori_loop(0, 32, tag_add, 0)

    pltpu.sync_copy(off_hbm.at[pl.ds(bag0, 400)], offbuf)
    p_start = offbuf[pl.ds(0, 16)][0]
    p_end = offbuf[pl.ds(BAGS_PER_W, 16)][0]
    p0 = (p_start // 8) * 8
    nch = (p_end - p0 + CH - 1) // CH

    ids2 = [idsbuf, idsbuf2]
    rows2 = [rows, rowsb]
    sem2 = [gsem, gsemb]

    def issue(kk, par):
        pltpu.sync_copy(cid_hbm.at[pl.ds(p0 + kk * CH, CH)], ids2[par])
        pltpu.async_copy(ctab_hbm.at[ids2[par]], rows2[par], sem2[par])

    @pl.when(nch > 0)
    def _():
        issue(0, 0)

    NQ = H // 16
    mbuf[pl.ds(CH, 16)] = jnp.full((16,), -1, jnp.int32)

    def chunk(kk, carry):
        base = p0 + kk * CH

        for j in range(CH // 16):
            mbuf[pl.ds(j * 16, 16)] = jnp.zeros((16,), jnp.int32)
        for j in range(NOFF):
            offv = offbuf[pl.ds(j * 16, 16)]
            offn = offbuf[pl.ds(j * 16 + 1, 16)]
            m = (offv >= base) & (offv < base + CH) & (offv < offn)
            plsc.store_scatter(mbuf, [offv - base], 16 * j + lanes, mask=m)

        def segfin(j, cr):
            pos = base + j * 16 + lanes
            mv = jnp.maximum(plsc.cummax(mbuf[pl.ds(j * 16, 16)]), cr)
            newcr = mv[15]
            valid = (pos >= p_start) & (pos < p_end)
            mbuf[pl.ds(j * 16, 16)] = jnp.where(valid, mv, TRASH)
            return newcr

        cr = lax.fori_loop(0, CH // 16, segfin, carry)

        par = lax.rem(kk, 2)

        @pl.when(kk + 1 < nch)
        def _():
            @pl.when(par == 0)
            def _():
                issue(kk + 1, 1)

            @pl.when(par == 1)
            def _():
                issue(kk + 1, 0)

        def rmw(p):
            pltpu.make_async_copy(
                ctab_hbm.at[pl.ds(0, CH)], rows2[p], sem2[p]).wait()

            zeros = tuple(jnp.zeros((16,), jnp.float32) for _ in range(NQ))

            def inner(j, accs):
                sv = mbuf[pl.ds(j * 16, 16)]
                nxt = mbuf[pl.ds(j * 16 + 1, 16)]
                lastv = (sv != nxt).astype(jnp.int32)
                for l in range(16):
                    t = j * 16 + l
                    half = []
                    for hh in range(H // 32):
                        vi = rows2[p][t, pl.ds(hh * 16, 16)]
                        half.append(plsc.bitcast(vi << 16, jnp.float32))
                        half.append(plsc.bitcast(vi & jnp.int32(-65536),
                                                 jnp.float32))
                    accs = tuple(a + h for a, h in zip(accs, half))
                    last = lastv[l] != 0
                    sg = sv[l]

                    @pl.when(last)
                    def _(accs=accs, sg=sg):
                        for q in range(NQ):
                            sl = pl.ds(q * 16, 16)
                            acc[sg, sl] = acc[sg, sl] + accs[q]

                    accs = tuple(jnp.where(last, z, a)
                                 for z, a in zip(zeros, accs))
                return accs

            lax.fori_loop(0, CH // 16, inner, zeros)

        @pl.when(par == 0)
        def _():
            rmw(0)

        @pl.when(par == 1)
        def _():
            rmw(1)

        return cr

    lax.fori_loop(0, nch, chunk, 0)

    pltpu.sync_copy(acc.at[pl.ds(0, BAGS_PER_W)],
                    x_hbm.at[pl.ds(bag0, BAGS_PER_W)])


_sc_embed_call = functools.partial(
    pl.kernel,
    out_type=jax.ShapeDtypeStruct((N_BAGS, H), jnp.float32),
    mesh=plsc.VectorSubcoreMesh(core_axis_name="c", subcore_axis_name="s"),
    compiler_params=pltpu.CompilerParams(needs_layout_passes=False),
    scratch_types=[
        pltpu.VMEM((CH,), jnp.int32),
        pltpu.VMEM((CH,), jnp.int32),
        pltpu.VMEM((32,), jnp.int32),
        pltpu.VMEM((CH, H // 2), jnp.int32),
        pltpu.VMEM((CH, H // 2), jnp.int32),
        pltpu.VMEM((400,), jnp.int32),
        pltpu.VMEM((CH + 16,), jnp.int32),
        pltpu.VMEM((BAGS_PER_W + 1, H), jnp.float32),
        pltpu.SemaphoreType.DMA,
        pltpu.SemaphoreType.DMA,
    ],
)(_sc_embed)


def _tc_classifier(x_ref, w_ref, b_ref, o_ref):
    y = jnp.dot(x_ref[...], w_ref[...], preferred_element_type=jnp.float32)
    y = jnp.maximum(y + b_ref[...], 0.0)
    y = y - jnp.max(y, axis=1, keepdims=True)
    o_ref[...] = jnp.exp(y)


def kernel(char_ids, offsets, prev_tag_ids, word_ids,
           char_table, tag_table, word_table, W_w, W_b):
    cid_pad = jnp.concatenate(
        [char_ids.astype(jnp.int32), jnp.zeros((CH,), jnp.int32)])
    off_pad = jnp.concatenate(
        [offsets.astype(jnp.int32), jnp.full((16,), TOTAL, jnp.int32)])
    wf = word_ids.reshape(-1).astype(jnp.int32)

    def _perm(t):
        v, h = t.shape
        tb = (t.astype(jnp.bfloat16).reshape(v, h // 32, 2, 16)
              .transpose(0, 1, 3, 2).reshape(v, h // 2, 2))
        return jax.lax.bitcast_convert_type(tb, jnp.int32)

    x = _sc_embed_call(cid_pad, off_pad, wf,
                       prev_tag_ids.astype(jnp.int32),
                       _perm(char_table), _perm(tag_table), word_table)
    x = x.reshape(B, 3 * H)

    blk = 512
    out = pl.pallas_call(
        _tc_classifier,
        grid=(B // blk,),
        in_specs=[
            pl.BlockSpec((blk, 3 * H), lambda i: (i, 0)),
            pl.BlockSpec((3 * H, TAGS), lambda i: (0, 0)),
            pl.BlockSpec((1, TAGS), lambda i: (0, 0)),
        ],
        out_specs=pl.BlockSpec((blk, TAGS), lambda i: (i, 0)),
        out_shape=jax.ShapeDtypeStruct((B, TAGS), jnp.float32),
    )(x, W_w.T, W_b.reshape(1, TAGS))
    return out
